# baseline (speedup 1.0000x reference)
import functools

import jax
import jax.numpy as jnp
from jax import lax
from jax.experimental import pallas as pl
from jax.experimental.pallas import tpu as pltpu
from jax.experimental.pallas import tpu_sc as plsc

N = 50000
D_X = 256
B = 128
D_U = 64
OUT_CH = 64

NW = 32
S = 1568
C = 112
NCHUNK = S // C
ACC_ROWS = 136
DUMMY = 128
LAST_START = N - S


def _sc_body(x_hbm, batch_hbm, part_x, part_c,
             xbuf0, xbuf1, segf, accx, accc, sx0, sx1, sv):
    nc = 2
    wid = lax.axis_index("s") * nc + lax.axis_index("c")
    start = jnp.minimum(wid * S, LAST_START)
    overlap = wid * S - start

    z16 = jnp.zeros((16,), jnp.float32)
    o16 = jnp.ones((16,), jnp.float32)
    s16 = jnp.full((16,), 16.0, jnp.float32)

    def _issue(i, buf, semx):
        pltpu.async_copy(x_hbm.at[pl.ds(start + i * C, C)], buf, semx)

    def _wait(i, buf, semx):
        pltpu.make_async_copy(
            x_hbm.at[pl.ds(start + i * C, C)], buf, semx).wait()

    pltpu.async_copy(batch_hbm.at[pl.ds(start, S)], segf, sv)
    _issue(0, xbuf0, sx0)
    _issue(1, xbuf1, sx1)

    def _zrow(r, carry):
        for c in range(D_X // 16):
            accx[r, pl.ds(c * 16, 16)] = z16
        accc[r, pl.ds(0, 16)] = z16
        return carry
    lax.fori_loop(0, ACC_ROWS, _zrow, 0)

    pltpu.make_async_copy(batch_hbm.at[pl.ds(start, S)], segf, sv).wait()

    lanes = lax.iota(jnp.int32, 16)

    def _process(i, buf):
        def _group(r, carry2):
            g0 = i * C + r * 16
            segs = segf[pl.ds(g0, 16)]
            segs = jnp.where(g0 + lanes < overlap, jnp.int32(DUMMY), segs)
            j0 = r * 16
            s_first = segs[0]
            s_last = segs[15]

            def _uniform():
                for c0 in range(0, D_X // 16, 4):
                    accs = [buf[j0, pl.ds((c0 + c) * 16, 16)]
                            for c in range(4)]
                    for l in range(1, 16):
                        for c in range(4):
                            accs[c] = accs[c] + buf[j0 + l,
                                                    pl.ds((c0 + c) * 16, 16)]
                    for c in range(4):
                        plsc.addupdate(
                            accx.at[s_first, pl.ds((c0 + c) * 16, 16)],
                            accs[c])
                plsc.addupdate(accc.at[s_first, pl.ds(0, 16)], s16)

            def _mixed():
                for l in range(16):
                    s = segs[l]
                    for c in range(D_X // 16):
                        plsc.addupdate(accx.at[s, pl.ds(c * 16, 16)],
                                       buf[j0 + l, pl.ds(c * 16, 16)])
                    plsc.addupdate(accc.at[s, pl.ds(0, 16)], o16)

            lax.cond(s_first == s_last, _uniform, _mixed)
            return carry2
        lax.fori_loop(0, C // 16, _group, 0)

    def _chunk(i, carry):
        for m, (buf, semx) in enumerate(((xbuf0, sx0), (xbuf1, sx1))):
            @pl.when(i % 2 == m)
            def _go():
                _wait(i, buf, semx)
                _process(i, buf)

                @pl.when(i + 2 < NCHUNK)
                def _next():
                    _issue(i + 2, buf, semx)
        return carry
    lax.fori_loop(0, NCHUNK, _chunk, 0)

    pltpu.sync_copy(accx, part_x.at[wid])
    pltpu.sync_copy(accc, part_c.at[wid])


_sc_segment_sums = functools.partial(
    pl.kernel,
    out_type=(
        jax.ShapeDtypeStruct((NW, ACC_ROWS, D_X), jnp.float32),
        jax.ShapeDtypeStruct((NW, ACC_ROWS, 16), jnp.float32),
    ),
    mesh=plsc.VectorSubcoreMesh(core_axis_name="c", subcore_axis_name="s",
                                num_cores=2, num_subcores=16),
    scratch_types=[
        pltpu.VMEM((C, D_X), jnp.float32),
        pltpu.VMEM((C, D_X), jnp.float32),
        pltpu.VMEM((S,), jnp.int32),
        pltpu.VMEM((ACC_ROWS, D_X), jnp.float32),
        pltpu.VMEM((ACC_ROWS, 16), jnp.float32),
        pltpu.SemaphoreType.DMA,
        pltpu.SemaphoreType.DMA,
        pltpu.SemaphoreType.DMA,
    ],
)(_sc_body)


def _mlp_body(px_ref, pc_ref, u_ref, w1_ref, b1_ref, w2_ref, b2_ref,
              w3_ref, b3_ref, o_ref):
    sums = jnp.sum(px_ref[...][:, :B, :], axis=0)
    cnts = jnp.sum(pc_ref[...][:, :B, 0:1], axis=0)
    mean = sums / jnp.maximum(cnts, 1.0)
    cat = jnp.concatenate([u_ref[...], mean], axis=1)
    h = lax.dot_general(cat, w1_ref[...], (((1,), (1,)), ((), ())),
                        preferred_element_type=jnp.float32)
    h = jnp.maximum(h + b1_ref[...][None, :], 0.0)
    h = lax.dot_general(h, w2_ref[...], (((1,), (1,)), ((), ())),
                        preferred_element_type=jnp.float32)
    h = jnp.maximum(h + b2_ref[...][None, :], 0.0)
    h = lax.dot_general(h, w3_ref[...], (((1,), (1,)), ((), ())),
                        preferred_element_type=jnp.float32)
    o_ref[...] = h + b3_ref[...][None, :]


_mlp_call = pl.pallas_call(
    _mlp_body,
    out_shape=jax.ShapeDtypeStruct((B, OUT_CH), jnp.float32),
)


def kernel(x, edge_index, edge_attr, u, batch, W1, b1, W2, b2, W3, b3):
    del edge_index, edge_attr
    part_x, part_c = _sc_segment_sums(x, batch.astype(jnp.int32))
    return _mlp_call(part_x, part_c, u, W1, b1, W2, b2, W3, b3)


# speedup vs baseline: 1.0013x; 1.0013x over previous
import functools

import jax
import jax.numpy as jnp
from jax import lax
from jax.experimental import pallas as pl
from jax.experimental.pallas import tpu as pltpu
from jax.experimental.pallas import tpu_sc as plsc

N = 50000
D_X = 256
B = 128
D_U = 64
OUT_CH = 64

NW = 32
S = 1568
C = 112
NCHUNK = S // C
ACC_ROWS = 136
DUMMY = 128
LAST_START = N - S


def _sc_body(x_hbm, batch_hbm, part_x, part_c,
             xbuf0, xbuf1, segf, accx, accc, sx0, sx1, sv):
    nc = 2
    wid = lax.axis_index("s") * nc + lax.axis_index("c")
    start = jnp.minimum(wid * S, LAST_START)
    overlap = wid * S - start

    z16 = jnp.zeros((16,), jnp.float32)
    o16 = jnp.ones((16,), jnp.float32)
    s16 = jnp.full((16,), 16.0, jnp.float32)

    def _issue(i, buf, semx):
        pltpu.async_copy(x_hbm.at[pl.ds(start + i * C, C)], buf, semx)

    def _wait(i, buf, semx):
        pltpu.make_async_copy(
            x_hbm.at[pl.ds(start + i * C, C)], buf, semx).wait()

    pltpu.async_copy(batch_hbm.at[pl.ds(start, S)], segf, sv)
    _issue(0, xbuf0, sx0)
    _issue(1, xbuf1, sx1)

    def _zrow(r, carry):
        for c in range(D_X // 16):
            accx[r, pl.ds(c * 16, 16)] = z16
        accc[r, pl.ds(0, 16)] = z16
        return carry
    lax.fori_loop(0, ACC_ROWS, _zrow, 0)

    pltpu.make_async_copy(batch_hbm.at[pl.ds(start, S)], segf, sv).wait()

    lanes = lax.iota(jnp.int32, 16)

    def _process(i, buf):
        def _group(r, carry2):
            g0 = i * C + r * 16
            segs = segf[pl.ds(g0, 16)]
            segs = jnp.where(g0 + lanes < overlap, jnp.int32(DUMMY), segs)
            j0 = r * 16
            s_first = segs[0]
            s_last = segs[15]

            def _uniform():
                for c0 in range(0, D_X // 16, 4):
                    accs = [buf[j0, pl.ds((c0 + c) * 16, 16)]
                            for c in range(4)]
                    for l in range(1, 16):
                        for c in range(4):
                            accs[c] = accs[c] + buf[j0 + l,
                                                    pl.ds((c0 + c) * 16, 16)]
                    for c in range(4):
                        plsc.addupdate(
                            accx.at[s_first, pl.ds((c0 + c) * 16, 16)],
                            accs[c])
                plsc.addupdate(accc.at[s_first, pl.ds(0, 16)], s16)

            def _mixed():
                for l in range(16):
                    s = segs[l]
                    for c in range(D_X // 16):
                        plsc.addupdate(accx.at[s, pl.ds(c * 16, 16)],
                                       buf[j0 + l, pl.ds(c * 16, 16)])
                    plsc.addupdate(accc.at[s, pl.ds(0, 16)], o16)

            lax.cond(s_first == s_last, _uniform, _mixed)
            return carry2
        lax.fori_loop(0, C // 16, _group, 0)

    def _chunk(i, carry):
        for m, (buf, semx) in enumerate(((xbuf0, sx0), (xbuf1, sx1))):
            @pl.when(i % 2 == m)
            def _go():
                _wait(i, buf, semx)
                _process(i, buf)

                @pl.when(i + 2 < NCHUNK)
                def _next():
                    _issue(i + 2, buf, semx)
        return carry
    lax.fori_loop(0, NCHUNK, _chunk, 0)

    pltpu.sync_copy(accx.at[pl.ds(0, B)], part_x.at[wid])
    pltpu.sync_copy(accc.at[pl.ds(0, B)], part_c.at[wid])


_sc_segment_sums = functools.partial(
    pl.kernel,
    out_type=(
        jax.ShapeDtypeStruct((NW, B, D_X), jnp.float32),
        jax.ShapeDtypeStruct((NW, B, 16), jnp.float32),
    ),
    mesh=plsc.VectorSubcoreMesh(core_axis_name="c", subcore_axis_name="s",
                                num_cores=2, num_subcores=16),
    scratch_types=[
        pltpu.VMEM((C, D_X), jnp.float32),
        pltpu.VMEM((C, D_X), jnp.float32),
        pltpu.VMEM((S,), jnp.int32),
        pltpu.VMEM((ACC_ROWS, D_X), jnp.float32),
        pltpu.VMEM((ACC_ROWS, 16), jnp.float32),
        pltpu.SemaphoreType.DMA,
        pltpu.SemaphoreType.DMA,
        pltpu.SemaphoreType.DMA,
    ],
)(_sc_body)


def _mlp_body(px_ref, pc_ref, u_ref, w1_ref, b1_ref, w2_ref, b2_ref,
              w3_ref, b3_ref, o_ref):
    sums = jnp.sum(px_ref[...], axis=0)
    cnts = jnp.sum(pc_ref[...][:, :, 0:1], axis=0)
    mean = sums / jnp.maximum(cnts, 1.0)
    cat = jnp.concatenate([u_ref[...], mean], axis=1)
    h = lax.dot_general(cat, w1_ref[...], (((1,), (1,)), ((), ())),
                        preferred_element_type=jnp.float32)
    h = jnp.maximum(h + b1_ref[...][None, :], 0.0)
    h = lax.dot_general(h, w2_ref[...], (((1,), (1,)), ((), ())),
                        preferred_element_type=jnp.float32)
    h = jnp.maximum(h + b2_ref[...][None, :], 0.0)
    h = lax.dot_general(h, w3_ref[...], (((1,), (1,)), ((), ())),
                        preferred_element_type=jnp.float32)
    o_ref[...] = h + b3_ref[...][None, :]


_mlp_call = pl.pallas_call(
    _mlp_body,
    out_shape=jax.ShapeDtypeStruct((B, OUT_CH), jnp.float32),
)


def kernel(x, edge_index, edge_attr, u, batch, W1, b1, W2, b2, W3, b3):
    del edge_index, edge_attr
    part_x, part_c = _sc_segment_sums(x, batch.astype(jnp.int32))
    return _mlp_call(part_x, part_c, u, W1, b1, W2, b2, W3, b3)


# speedup vs baseline: 1.0037x; 1.0024x over previous
import functools

import jax
import jax.numpy as jnp
from jax import lax
from jax.experimental import pallas as pl
from jax.experimental.pallas import tpu as pltpu
from jax.experimental.pallas import tpu_sc as plsc

N = 50000
D_X = 256
B = 128
D_U = 64
OUT_CH = 64

NW = 32
S = 1568
C = 112
NCHUNK = S // C
ACC_ROWS = 136
DUMMY = 128
LAST_START = N - S


def _sc_body(x_hbm, batch_hbm, part_x, part_c,
             xbuf0, xbuf1, segf, accx, accc, sx0, sx1, sv):
    nc = 2
    wid = lax.axis_index("s") * nc + lax.axis_index("c")
    start = jnp.minimum(wid * S, LAST_START)
    overlap = wid * S - start

    z16 = jnp.zeros((16,), jnp.float32)
    o16 = jnp.ones((16,), jnp.float32)
    s16 = jnp.full((16,), 16.0, jnp.float32)

    def _issue(i, buf, semx):
        pltpu.async_copy(x_hbm.at[pl.ds(start + i * C, C)], buf, semx)

    def _wait(i, buf, semx):
        pltpu.make_async_copy(
            x_hbm.at[pl.ds(start + i * C, C)], buf, semx).wait()

    pltpu.async_copy(batch_hbm.at[pl.ds(start, S)], segf, sv)
    _issue(0, xbuf0, sx0)
    _issue(1, xbuf1, sx1)

    def _zrow(r, carry):
        for c in range(D_X // 16):
            accx[r, pl.ds(c * 16, 16)] = z16
        accc[r, pl.ds(0, 16)] = z16
        return carry
    lax.fori_loop(0, ACC_ROWS, _zrow, 0)

    pltpu.make_async_copy(batch_hbm.at[pl.ds(start, S)], segf, sv).wait()

    lanes = lax.iota(jnp.int32, 16)

    def _process(i, buf):
        def _group(r, carry2):
            g0 = i * C + r * 16
            segs = segf[pl.ds(g0, 16)]
            segs = jnp.where(g0 + lanes < overlap, jnp.int32(DUMMY), segs)
            j0 = r * 16
            s_first = segs[0]
            s_last = segs[15]

            def _uniform():
                for c0 in range(0, D_X // 16, 4):
                    accs = [buf[j0, pl.ds((c0 + c) * 16, 16)]
                            for c in range(4)]
                    for l in range(1, 16):
                        for c in range(4):
                            accs[c] = accs[c] + buf[j0 + l,
                                                    pl.ds((c0 + c) * 16, 16)]
                    for c in range(4):
                        plsc.addupdate(
                            accx.at[s_first, pl.ds((c0 + c) * 16, 16)],
                            accs[c])
                plsc.addupdate(accc.at[s_first, pl.ds(0, 16)], s16)

            def _mixed():
                for l in range(16):
                    s = segs[l]
                    for c in range(D_X // 16):
                        plsc.addupdate(accx.at[s, pl.ds(c * 16, 16)],
                                       buf[j0 + l, pl.ds(c * 16, 16)])
                    plsc.addupdate(accc.at[s, pl.ds(0, 16)], o16)

            lax.cond(s_first == s_last, _uniform, _mixed)
            return carry2
        lax.fori_loop(0, C // 16, _group, 0)

    def _pair(p, carry):
        i0 = 2 * p
        i1 = i0 + 1
        _wait(i0, xbuf0, sx0)
        _process(i0, xbuf0)

        @pl.when(p < NCHUNK // 2 - 1)
        def _prefetch0():
            _issue(i0 + 2, xbuf0, sx0)

        _wait(i1, xbuf1, sx1)
        _process(i1, xbuf1)

        @pl.when(p < NCHUNK // 2 - 1)
        def _prefetch1():
            _issue(i1 + 2, xbuf1, sx1)
        return carry
    lax.fori_loop(0, NCHUNK // 2, _pair, 0)

    pltpu.sync_copy(accx.at[pl.ds(0, B)], part_x.at[wid])
    pltpu.sync_copy(accc.at[pl.ds(0, B)], part_c.at[wid])


_sc_segment_sums = functools.partial(
    pl.kernel,
    out_type=(
        jax.ShapeDtypeStruct((NW, B, D_X), jnp.float32),
        jax.ShapeDtypeStruct((NW, B, 16), jnp.float32),
    ),
    mesh=plsc.VectorSubcoreMesh(core_axis_name="c", subcore_axis_name="s",
                                num_cores=2, num_subcores=16),
    scratch_types=[
        pltpu.VMEM((C, D_X), jnp.float32),
        pltpu.VMEM((C, D_X), jnp.float32),
        pltpu.VMEM((S,), jnp.int32),
        pltpu.VMEM((ACC_ROWS, D_X), jnp.float32),
        pltpu.VMEM((ACC_ROWS, 16), jnp.float32),
        pltpu.SemaphoreType.DMA,
        pltpu.SemaphoreType.DMA,
        pltpu.SemaphoreType.DMA,
    ],
)(_sc_body)


def _mlp_body(px_ref, pc_ref, u_ref, w1_ref, b1_ref, w2_ref, b2_ref,
              w3_ref, b3_ref, o_ref):
    sums = jnp.sum(px_ref[...], axis=0)
    cnts = jnp.sum(pc_ref[...][:, :, 0:1], axis=0)
    mean = sums / jnp.maximum(cnts, 1.0)
    cat = jnp.concatenate([u_ref[...], mean], axis=1)
    h = lax.dot_general(cat, w1_ref[...], (((1,), (1,)), ((), ())),
                        preferred_element_type=jnp.float32)
    h = jnp.maximum(h + b1_ref[...][None, :], 0.0)
    h = lax.dot_general(h, w2_ref[...], (((1,), (1,)), ((), ())),
                        preferred_element_type=jnp.float32)
    h = jnp.maximum(h + b2_ref[...][None, :], 0.0)
    h = lax.dot_general(h, w3_ref[...], (((1,), (1,)), ((), ())),
                        preferred_element_type=jnp.float32)
    o_ref[...] = h + b3_ref[...][None, :]


_mlp_call = pl.pallas_call(
    _mlp_body,
    out_shape=jax.ShapeDtypeStruct((B, OUT_CH), jnp.float32),
)


def kernel(x, edge_index, edge_attr, u, batch, W1, b1, W2, b2, W3, b3):
    del edge_index, edge_attr
    part_x, part_c = _sc_segment_sums(x, batch.astype(jnp.int32))
    return _mlp_call(part_x, part_c, u, W1, b1, W2, b2, W3, b3)


# speedup vs baseline: 1.0177x; 1.0140x over previous
import functools

import jax
import jax.numpy as jnp
from jax import lax
from jax.experimental import pallas as pl
from jax.experimental.pallas import tpu as pltpu
from jax.experimental.pallas import tpu_sc as plsc

N = 50000
D_X = 256
B = 128
D_U = 64
OUT_CH = 64

NW = 32
S = 1568
C = 112
NCHUNK = S // C
ACC_ROWS = 136
DUMMY = 128
LAST_START = N - S


def _sc_body(x_hbm, batch_hbm, part_x, part_c,
             xbuf0, xbuf1, segf, accx, accc, sx0, sx1, sv):
    nc = 2
    wid = lax.axis_index("s") * nc + lax.axis_index("c")
    start = jnp.minimum(wid * S, LAST_START)
    overlap = wid * S - start

    z16 = jnp.zeros((16,), jnp.float32)
    o16 = jnp.ones((16,), jnp.float32)
    s16 = jnp.full((16,), 16.0, jnp.float32)

    def _issue(i, buf, semx):
        pltpu.async_copy(x_hbm.at[pl.ds(start + i * C, C)], buf, semx)

    def _wait(i, buf, semx):
        pltpu.make_async_copy(
            x_hbm.at[pl.ds(start + i * C, C)], buf, semx).wait()

    pltpu.async_copy(batch_hbm.at[pl.ds(start, S)], segf, sv)
    _issue(0, xbuf0, sx0)

    def _zrow(r, carry):
        for c in range(D_X // 16):
            accx[r, pl.ds(c * 16, 16)] = z16
        accc[r, pl.ds(0, 16)] = z16
        return carry
    lax.fori_loop(0, ACC_ROWS, _zrow, 0)

    pltpu.make_async_copy(batch_hbm.at[pl.ds(start, S)], segf, sv).wait()

    lanes = lax.iota(jnp.int32, 16)

    def _process(i, buf):
        def _group(r, carry2):
            g0 = i * C + r * 16
            segs = segf[pl.ds(g0, 16)]
            segs = jnp.where(g0 + lanes < overlap, jnp.int32(DUMMY), segs)
            j0 = r * 16
            s_first = segs[0]
            s_last = segs[15]

            def _uniform():
                for c0 in range(0, D_X // 16, 4):
                    accs = [buf[j0, pl.ds((c0 + c) * 16, 16)]
                            for c in range(4)]
                    for l in range(1, 16):
                        for c in range(4):
                            accs[c] = accs[c] + buf[j0 + l,
                                                    pl.ds((c0 + c) * 16, 16)]
                    for c in range(4):
                        plsc.addupdate(
                            accx.at[s_first, pl.ds((c0 + c) * 16, 16)],
                            accs[c])
                plsc.addupdate(accc.at[s_first, pl.ds(0, 16)], s16)

            def _mixed():
                for l in range(16):
                    s = segs[l]
                    for c in range(D_X // 16):
                        plsc.addupdate(accx.at[s, pl.ds(c * 16, 16)],
                                       buf[j0 + l, pl.ds(c * 16, 16)])
                    plsc.addupdate(accc.at[s, pl.ds(0, 16)], o16)

            lax.cond(s_first == s_last, _uniform, _mixed)
            return carry2
        lax.fori_loop(0, C // 16, _group, 0)

    def _pair(p, carry):
        i0 = 2 * p
        i1 = i0 + 1
        _issue(i1, xbuf1, sx1)
        _wait(i0, xbuf0, sx0)
        _process(i0, xbuf0)

        @pl.when(p < NCHUNK // 2 - 1)
        def _prefetch():
            _issue(i0 + 2, xbuf0, sx0)

        _wait(i1, xbuf1, sx1)
        _process(i1, xbuf1)
        return carry
    lax.fori_loop(0, NCHUNK // 2, _pair, 0)

    pltpu.sync_copy(accx.at[pl.ds(0, B)], part_x.at[wid])
    pltpu.sync_copy(accc.at[pl.ds(0, B)], part_c.at[wid])


_sc_segment_sums = functools.partial(
    pl.kernel,
    out_type=(
        jax.ShapeDtypeStruct((NW, B, D_X), jnp.float32),
        jax.ShapeDtypeStruct((NW, B, 16), jnp.float32),
    ),
    mesh=plsc.VectorSubcoreMesh(core_axis_name="c", subcore_axis_name="s",
                                num_cores=2, num_subcores=16),
    scratch_types=[
        pltpu.VMEM((C, D_X), jnp.float32),
        pltpu.VMEM((C, D_X), jnp.float32),
        pltpu.VMEM((S,), jnp.int32),
        pltpu.VMEM((ACC_ROWS, D_X), jnp.float32),
        pltpu.VMEM((ACC_ROWS, 16), jnp.float32),
        pltpu.SemaphoreType.DMA,
        pltpu.SemaphoreType.DMA,
        pltpu.SemaphoreType.DMA,
    ],
)(_sc_body)


def _mlp_body(px_ref, pc_ref, u_ref, w1_ref, b1_ref, w2_ref, b2_ref,
              w3_ref, b3_ref, o_ref):
    sums = jnp.sum(px_ref[...], axis=0)
    cnts = jnp.sum(pc_ref[...][:, :, 0:1], axis=0)
    mean = sums / jnp.maximum(cnts, 1.0)
    cat = jnp.concatenate([u_ref[...], mean], axis=1)
    h = lax.dot_general(cat, w1_ref[...], (((1,), (1,)), ((), ())),
                        preferred_element_type=jnp.float32)
    h = jnp.maximum(h + b1_ref[...][None, :], 0.0)
    h = lax.dot_general(h, w2_ref[...], (((1,), (1,)), ((), ())),
                        preferred_element_type=jnp.float32)
    h = jnp.maximum(h + b2_ref[...][None, :], 0.0)
    h = lax.dot_general(h, w3_ref[...], (((1,), (1,)), ((), ())),
                        preferred_element_type=jnp.float32)
    o_ref[...] = h + b3_ref[...][None, :]


_mlp_call = pl.pallas_call(
    _mlp_body,
    out_shape=jax.ShapeDtypeStruct((B, OUT_CH), jnp.float32),
)


def kernel(x, edge_index, edge_attr, u, batch, W1, b1, W2, b2, W3, b3):
    del edge_index, edge_attr
    part_x, part_c = _sc_segment_sums(x, batch.astype(jnp.int32))
    return _mlp_call(part_x, part_c, u, W1, b1, W2, b2, W3, b3)
